# Initial kernel scaffold; baseline (speedup 1.0000x reference)
import jax
import jax.numpy as jnp
from jax.experimental import pallas as pl


def kernel(x, Wq, bq, Wk, bk, position_bias):
    raise NotImplementedError("write your pallas kernel here")


# speedup vs baseline: 7.6428x; 7.6428x over previous
import functools

import jax
import jax.numpy as jnp
from jax import lax
from jax.experimental import pallas as pl

DIM = 96
P = 1024
K = 16
TEMP = 0.1
B = 32
NEG = -1e9


def _router_body(x_ref, wq_ref, bq_ref, wk_ref, bk_ref, bias_ref,
                 routes_ref, w_ref):
    xp = x_ref[0, 1:, :]
    wq = wq_ref[...]
    wk = wk_ref[...]
    q = jnp.dot(xp, wq.T, preferred_element_type=jnp.float32) + bq_ref[0]
    k = jnp.dot(xp, wk.T, preferred_element_type=jnp.float32) + bk_ref[0]
    qn = q / jnp.maximum(jnp.sqrt(jnp.sum(q * q, axis=-1, keepdims=True)), 1e-12)
    kn = k / jnp.maximum(jnp.sqrt(jnp.sum(k * k, axis=-1, keepdims=True)), 1e-12)
    s = jnp.dot(qn, kn.T, preferred_element_type=jnp.float32) + bias_ref[...]
    row = lax.broadcasted_iota(jnp.int32, (P, P), 0)
    col = lax.broadcasted_iota(jnp.int32, (P, P), 1)
    s = jnp.where(row == col, NEG, s)

    vals = []
    idxs = []
    for _ in range(K):
        m = jnp.max(s, axis=1, keepdims=True)
        idx = jnp.min(jnp.where(s == m, col, P), axis=1)
        vals.append(m[:, 0])
        idxs.append(idx)
        s = jnp.where(col == idx[:, None], NEG, s)

    tv = jnp.stack(vals, axis=1) * (1.0 / TEMP)
    e = jnp.exp(tv - tv[:, 0:1])
    w_ref[0] = e / jnp.sum(e, axis=1, keepdims=True)
    routes_ref[0] = jnp.stack(idxs, axis=1)


@functools.partial(jax.jit, static_argnums=())
def kernel(x, Wq, bq, Wk, bk, position_bias):
    bq2 = bq.reshape(1, DIM)
    bk2 = bk.reshape(1, DIM)
    grid = (B,)
    routes, weights = pl.pallas_call(
        _router_body,
        grid=grid,
        in_specs=[
            pl.BlockSpec((1, P + 1, DIM), lambda b: (b, 0, 0)),
            pl.BlockSpec((DIM, DIM), lambda b: (0, 0)),
            pl.BlockSpec((1, DIM), lambda b: (0, 0)),
            pl.BlockSpec((DIM, DIM), lambda b: (0, 0)),
            pl.BlockSpec((1, DIM), lambda b: (0, 0)),
            pl.BlockSpec((P, P), lambda b: (0, 0)),
        ],
        out_specs=[
            pl.BlockSpec((1, P, K), lambda b: (b, 0, 0)),
            pl.BlockSpec((1, P, K), lambda b: (b, 0, 0)),
        ],
        out_shape=[
            jax.ShapeDtypeStruct((B, P, K), jnp.int32),
            jax.ShapeDtypeStruct((B, P, K), jnp.float32),
        ],
    )(x, Wq, bq2, Wk, bk2, position_bias)
    return routes, weights
